# baseline (speedup 1.0000x reference)
import functools

import jax
import jax.numpy as jnp
from jax import lax
from jax.experimental import pallas as pl
from jax.experimental.pallas import tpu as pltpu
from jax.experimental.pallas import tpu_sc as plsc

_N_LOCATIONS = 16384
_DIM = 64
_SIDE_BITS = 7
_MAX_DEPTH = 7
_BATCH = 4096
_N_LEAVES = 4 ** _MAX_DEPTH


def _query_index(loc):
    x = loc & (2 ** _SIDE_BITS - 1)
    y = loc >> _SIDE_BITS
    c7 = 2 * (y & 1) + (x & 1)
    q6 = jnp.zeros_like(loc)
    for u in range(1, _SIDE_BITS):
        q6 = q6 + ((2 * ((y >> u) & 1) + ((x >> u) & 1)) << (2 * (_SIDE_BITS - 1 - u)))
    row = ((c7 >> 1) << (2 * (_SIDE_BITS - 1))) + q6
    half = (c7 & 1) * _DIM
    return row, half


def _expand_kernel(loc_ref, table_ref, W_ref, b_ref, out_ref, row_ref, half_ref):
    row, half = _query_index(loc_ref[:, :])
    row_ref[:, :] = row
    half_ref[:, :] = half
    s = table_ref[0:1, :]
    for d in range(_MAX_DEPTH - 1):
        y = jnp.dot(s, W_ref[d], preferred_element_type=jnp.float32)
        y = y + b_ref[d][None, :]
        s = jnp.concatenate(
            [y[:, _DIM * c:_DIM * (c + 1)] for c in range(4)], axis=0)
    y = jnp.dot(s, W_ref[_MAX_DEPTH - 1], preferred_element_type=jnp.float32)
    y = y + b_ref[_MAX_DEPTH - 1][None, :]
    half_rows = _N_LEAVES // 4
    out_ref[0:half_rows, :] = y[:, 0:2 * _DIM]
    out_ref[half_rows:2 * half_rows, :] = y[:, 2 * _DIM:4 * _DIM]


def _expand_states(location, table, W, b):
    return pl.pallas_call(
        _expand_kernel,
        out_shape=(
            jax.ShapeDtypeStruct((_N_LEAVES // 2, 2 * _DIM), jnp.float32),
            jax.ShapeDtypeStruct((_BATCH // 128, 128), jnp.int32),
            jax.ShapeDtypeStruct((_BATCH // 128, 128), jnp.int32),
        ),
    )(location.reshape(_BATCH // 128, 128), table, W, b)


_SC_INFO = plsc.get_sparse_core_info()
_NC = _SC_INFO.num_cores
_NW = _NC * _SC_INFO.num_subcores
_B_PER_W = _BATCH // _NW
_LANES = 16


@functools.partial(
    pl.kernel,
    mesh=plsc.VectorSubcoreMesh(core_axis_name="c", subcore_axis_name="s"),
    out_type=jax.ShapeDtypeStruct((_BATCH // 2, 2 * _DIM), jnp.float32),
    scratch_types=[
        pltpu.VMEM((_B_PER_W,), jnp.int32),
        pltpu.VMEM((_B_PER_W,), jnp.int32),
        pltpu.VMEM((_B_PER_W, 2 * _DIM), jnp.float32),
        pltpu.VMEM((_B_PER_W // 2, 2 * _DIM), jnp.float32),
        pltpu.SemaphoreType.DMA,
    ],
    compiler_params=pltpu.CompilerParams(needs_layout_passes=False),
)
def _sc_gather(row_hbm, half_hbm, pairs_hbm, out_hbm,
               row_v, half_v, rows_v, out_v, sem):
    wid = lax.axis_index("s") * _NC + lax.axis_index("c")
    base = wid * _B_PER_W
    pltpu.sync_copy(row_hbm.at[pl.ds(base, _B_PER_W)], row_v)
    pltpu.sync_copy(half_hbm.at[pl.ds(base, _B_PER_W)], half_v)
    pltpu.async_copy(pairs_hbm.at[row_v], rows_v, sem).wait()
    iota = lax.iota(jnp.int32, _LANES)
    for g in range(_B_PER_W // _LANES):
        jvec = iota + g * _LANES
        h16 = half_v[pl.ds(g * _LANES, _LANES)]
        orow = jvec >> 1
        ocol = (jvec & 1) * _DIM
        for dd in range(_DIM):
            vals = plsc.load_gather(rows_v, [jvec, h16 + dd])
            plsc.store_scatter(out_v, [orow, ocol + dd], vals)
    pltpu.sync_copy(out_v, out_hbm.at[pl.ds(wid * (_B_PER_W // 2), _B_PER_W // 2)])


def kernel(location, table, W, b):
    pairs, row, half = _expand_states(location, table, W, b)
    packed = _sc_gather(row.reshape(_BATCH), half.reshape(_BATCH), pairs)
    return packed.reshape(_BATCH, _DIM)


# speedup vs baseline: 1.4730x; 1.4730x over previous
import functools

import jax
import jax.numpy as jnp
from jax import lax
from jax.experimental import pallas as pl
from jax.experimental.pallas import tpu as pltpu
from jax.experimental.pallas import tpu_sc as plsc

_N_LOCATIONS = 16384
_DIM = 64
_SIDE_BITS = 7
_MAX_DEPTH = 7
_BATCH = 4096
_N_LEAVES = 4 ** _MAX_DEPTH


def _query_index(loc):
    x = loc & (2 ** _SIDE_BITS - 1)
    y = loc >> _SIDE_BITS
    c7 = 2 * (y & 1) + (x & 1)
    q6 = jnp.zeros_like(loc)
    for u in range(1, _SIDE_BITS):
        q6 = q6 + ((2 * ((y >> u) & 1) + ((x >> u) & 1)) << (2 * (_SIDE_BITS - 1 - u)))
    return ((c7 >> 1) << 13) + 2 * q6 + (c7 & 1)


def _expand_kernel(loc_ref, table_ref, W_ref, b_ref, out_ref, idx_ref):
    idx_ref[:, :] = _query_index(loc_ref[:, :])
    s = table_ref[0:1, :]
    for d in range(_MAX_DEPTH - 1):
        y = jnp.dot(s, W_ref[d], preferred_element_type=jnp.float32)
        y = y + b_ref[d][None, :]
        s = jnp.concatenate(
            [y[:, _DIM * c:_DIM * (c + 1)] for c in range(4)], axis=0)
    y = jnp.dot(s, W_ref[_MAX_DEPTH - 1], preferred_element_type=jnp.float32)
    y = y + b_ref[_MAX_DEPTH - 1][None, :]
    half_rows = _N_LEAVES // 4
    out_ref[0:half_rows, :] = y[:, 0:2 * _DIM]
    out_ref[half_rows:2 * half_rows, :] = y[:, 2 * _DIM:4 * _DIM]


def _expand_states(location, table, W, b):
    return pl.pallas_call(
        _expand_kernel,
        out_shape=(
            jax.ShapeDtypeStruct((_N_LEAVES // 2, 2 * _DIM), jnp.float32),
            jax.ShapeDtypeStruct((_BATCH // 128, 128), jnp.int32),
        ),
    )(location.reshape(_BATCH // 128, 128), table, W, b)


_SC_INFO = plsc.get_sparse_core_info()
_NC = _SC_INFO.num_cores
_NW = _NC * _SC_INFO.num_subcores
_B_PER_W = _BATCH // _NW


@functools.partial(
    pl.kernel,
    mesh=plsc.VectorSubcoreMesh(core_axis_name="c", subcore_axis_name="s"),
    out_type=jax.ShapeDtypeStruct((_BATCH, _DIM), jnp.float32),
    scratch_types=[
        pltpu.VMEM((_B_PER_W,), jnp.int32),
        pltpu.VMEM((_B_PER_W, _DIM), jnp.float32),
        pltpu.SemaphoreType.DMA,
    ],
    compiler_params=pltpu.CompilerParams(use_tc_tiling_on_sc=False),
)
def _sc_gather(idx_hbm, leaf_hbm, out_hbm, idx_v, rows_v, sem):
    wid = lax.axis_index("s") * _NC + lax.axis_index("c")
    base = wid * _B_PER_W
    pltpu.sync_copy(idx_hbm.at[pl.ds(base, _B_PER_W)], idx_v)
    pltpu.async_copy(leaf_hbm.at[idx_v], rows_v, sem).wait()
    pltpu.sync_copy(rows_v, out_hbm.at[pl.ds(base, _B_PER_W)])


def kernel(location, table, W, b):
    pairs, idx = _expand_states(location, table, W, b)
    leaf = pairs.reshape(_N_LEAVES, _DIM)
    return _sc_gather(idx.reshape(_BATCH), leaf)


# speedup vs baseline: 1.4757x; 1.0018x over previous
import functools

import jax
import jax.numpy as jnp
from jax import lax
from jax.experimental import pallas as pl
from jax.experimental.pallas import tpu as pltpu
from jax.experimental.pallas import tpu_sc as plsc

_N_LOCATIONS = 16384
_DIM = 64
_SIDE_BITS = 7
_MAX_DEPTH = 7
_BATCH = 4096
_N_LEAVES = 4 ** _MAX_DEPTH


def _query_index(loc):
    x = loc & (2 ** _SIDE_BITS - 1)
    y = loc >> _SIDE_BITS
    c7 = 2 * (y & 1) + (x & 1)
    q6 = jnp.zeros_like(loc)
    for u in range(1, _SIDE_BITS):
        q6 = q6 + ((2 * ((y >> u) & 1) + ((x >> u) & 1)) << (2 * (_SIDE_BITS - 1 - u)))
    return ((c7 >> 1) << 13) + 2 * q6 + (c7 & 1)


def _expand_kernel(loc_ref, table_ref, W_ref, b_ref, out_ref, idx_ref):
    idx_ref[:, :] = _query_index(loc_ref[:, :])
    s = table_ref[0:1, :]
    for d in range(_MAX_DEPTH - 1):
        y = jnp.dot(s, W_ref[d], preferred_element_type=jnp.float32)
        y = y + b_ref[d][None, :]
        s = jnp.concatenate(
            [y[:, _DIM * c:_DIM * (c + 1)] for c in range(4)], axis=0)
    y = jnp.dot(s, W_ref[_MAX_DEPTH - 1], preferred_element_type=jnp.float32)
    y = y + b_ref[_MAX_DEPTH - 1][None, :]
    half_rows = _N_LEAVES // 4
    out_ref[0:half_rows, :] = y[:, 0:2 * _DIM]
    out_ref[half_rows:2 * half_rows, :] = y[:, 2 * _DIM:4 * _DIM]


def _expand_states(location, table, W, b):
    return pl.pallas_call(
        _expand_kernel,
        out_shape=(
            jax.ShapeDtypeStruct((_N_LEAVES // 2, 2 * _DIM), jnp.float32),
            jax.ShapeDtypeStruct((_BATCH // 128, 128), jnp.int32),
        ),
    )(location.reshape(_BATCH // 128, 128), table, W, b)


_SC_INFO = plsc.get_sparse_core_info()
_NC = _SC_INFO.num_cores
_NW = _NC * _SC_INFO.num_subcores
_B_PER_W = _BATCH // _NW


@functools.partial(
    pl.kernel,
    mesh=plsc.VectorSubcoreMesh(core_axis_name="c", subcore_axis_name="s"),
    out_type=jax.ShapeDtypeStruct((_BATCH, _DIM), jnp.float32),
    scratch_types=[
        pltpu.VMEM((_B_PER_W,), jnp.int32),
        pltpu.VMEM((_B_PER_W, _DIM), jnp.float32),
        pltpu.SemaphoreType.DMA,
    ],
    compiler_params=pltpu.CompilerParams(use_tc_tiling_on_sc=False),
)
def _sc_gather(idx_hbm, leaf_hbm, out_hbm, idx_v, rows_v, sem):
    wid = lax.axis_index("s") * _NC + lax.axis_index("c")
    pltpu.sync_copy(idx_hbm.at[wid], idx_v)
    pltpu.async_copy(leaf_hbm.at[idx_v], rows_v, sem).wait()
    pltpu.sync_copy(rows_v, out_hbm.at[pl.ds(wid * _B_PER_W, _B_PER_W)])


def kernel(location, table, W, b):
    pairs, idx = _expand_states(location, table, W, b)
    leaf = pairs.reshape(_N_LEAVES, _DIM)
    return _sc_gather(idx, leaf)
